# baseline (speedup 1.0000x reference)
import functools

import jax
import jax.numpy as jnp
from jax import lax
from jax.experimental import pallas as pl
from jax.experimental.pallas import tpu as pltpu
from jax.experimental.pallas import tpu_sc as plsc

EMBED = 32
N_PIECE = 13
N_PIECE_PAD = 16
N_POS = 64
BATCH = 16384
BCHUNK = 1024
BQUARTER = BATCH // 4
NCH = BQUARTER // BCHUNK
LANES = 16
TROW = EMBED * N_PIECE_PAD

_DNUMS = lax.GatherDimensionNumbers(
    offset_dims=(), collapsed_slice_dims=(0,), start_index_map=(0,)
)


def _dyn_gather(vec, idx):
    return lax.gather(
        vec,
        idx.reshape(LANES, 1),
        dimension_numbers=_DNUMS,
        slice_sizes=(1,),
        mode=lax.GatherScatterMode.PROMISE_IN_BOUNDS,
    )


def _sc_lookup(board_t, piece_pad_t, pos_t):
    mesh = plsc.VectorSubcoreMesh(core_axis_name="c", subcore_axis_name="s")

    @functools.partial(
        pl.kernel,
        out_type=jax.ShapeDtypeStruct((8, 8, EMBED, BATCH), jnp.float32),
        mesh=mesh,
        compiler_params=pltpu.CompilerParams(use_tc_tiling_on_sc=True),
        scratch_types=[
            pltpu.VMEM((EMBED, N_PIECE_PAD), jnp.float32),
            pltpu.VMEM((EMBED, N_POS), jnp.float32),
            pltpu.VMEM((8 * TROW,), jnp.float32),
            pltpu.VMEM((8, BCHUNK), jnp.int32),
            pltpu.VMEM((EMBED, BCHUNK), jnp.float32),
            pltpu.VMEM((EMBED, BCHUNK), jnp.float32),
            pltpu.SemaphoreType.DMA,
            pltpu.SemaphoreType.DMA,
        ],
    )
    def k(board_hbm, piece_hbm, pos_hbm, out_hbm,
          pbuf, qbuf, tbuf, bb, ob0, ob1, os0, os1):
        wid = lax.axis_index("s") * 2 + lax.axis_index("c")
        r = wid // 4
        bq = wid % 4
        bbase = bq * BQUARTER

        off = (r // 2) * LANES
        pltpu.sync_copy(piece_hbm, pbuf)
        pltpu.sync_copy(pos_hbm, qbuf)
        lane0 = r * 8 - off
        lanes = [
            jnp.full((LANES,), 0, jnp.int32) + (lane0 + c) for c in range(8)
        ]

        def build_row(d, carry):
            piece_vec = pbuf[d, :]
            pos_vec = qbuf[d, pl.ds(off, LANES)]
            for c in range(8):
                tbuf[pl.ds(c * TROW + d * N_PIECE_PAD, LANES)] = (
                    piece_vec + _dyn_gather(pos_vec, lanes[c])
                )
            return carry

        lax.fori_loop(0, EMBED, build_row, 0)

        def fill_chunk(ob, c):
            @plsc.parallel_loop(0, BCHUNK // LANES, unroll=2)
            def _fill(kk):
                bv = bb[c, pl.ds(kk * LANES, LANES)]
                for d in range(EMBED):
                    tv = tbuf[pl.ds(c * TROW + d * N_PIECE_PAD, LANES)]
                    ob[d, pl.ds(kk * LANES, LANES)] = _dyn_gather(tv, bv)

        def out_slice(c, g):
            return out_hbm.at[r, c, :, pl.ds(bbase + g * BCHUNK, BCHUNK)]

        obs = (ob0, ob1)
        sems = (os0, os1)

        def loop_body(g, carry):
            pltpu.sync_copy(
                board_hbm.at[r, :, pl.ds(bbase + g * BCHUNK, BCHUNK)], bb
            )
            for c in range(8):
                buf = c % 2

                def _wait():
                    pltpu.make_async_copy(obs[buf], out_slice(0, 0), sems[buf]).wait()

                if c < 2:
                    pl.when(g > 0)(_wait)
                else:
                    _wait()
                fill_chunk(obs[buf], c)
                pltpu.async_copy(obs[buf], out_slice(c, g), sems[buf])
            return carry

        lax.fori_loop(0, NCH, loop_body, 0)
        pltpu.make_async_copy(ob0, out_slice(0, 0), os0).wait()
        pltpu.make_async_copy(ob1, out_slice(0, 0), os1).wait()

    return k(board_t, piece_pad_t, pos_t)


def kernel(board, piece_table, position_table):
    board_t = jnp.transpose(board.astype(jnp.int32), (1, 2, 0))
    piece_pad_t = jnp.pad(piece_table, ((0, N_PIECE_PAD - N_PIECE), (0, 0))).T
    pos_t = position_table.T
    out_t = _sc_lookup(board_t, piece_pad_t, pos_t)
    return jnp.transpose(out_t, (3, 0, 1, 2))


# speedup vs baseline: 1.1273x; 1.1273x over previous
import functools

import jax
import jax.numpy as jnp
from jax import lax
from jax.experimental import pallas as pl
from jax.experimental.pallas import tpu as pltpu
from jax.experimental.pallas import tpu_sc as plsc

EMBED = 32
N_PIECE = 13
N_PIECE_PAD = 16
N_POS = 64
BATCH = 16384
BCHUNK = 1024
BQUARTER = BATCH // 4
NCH = BQUARTER // BCHUNK
LANES = 16
TROW = EMBED * N_PIECE_PAD

_DNUMS = lax.GatherDimensionNumbers(
    offset_dims=(), collapsed_slice_dims=(0,), start_index_map=(0,)
)


def _dyn_gather(vec, idx):
    return lax.gather(
        vec,
        idx.reshape(LANES, 1),
        dimension_numbers=_DNUMS,
        slice_sizes=(1,),
        mode=lax.GatherScatterMode.PROMISE_IN_BOUNDS,
    )


def _sc_lookup(board_t, piece_pad_t, pos_t):
    mesh = plsc.VectorSubcoreMesh(core_axis_name="c", subcore_axis_name="s")

    @functools.partial(
        pl.kernel,
        out_type=jax.ShapeDtypeStruct((8, 8, EMBED, BATCH), jnp.float32),
        mesh=mesh,
        compiler_params=pltpu.CompilerParams(use_tc_tiling_on_sc=True),
        scratch_types=[
            pltpu.VMEM((EMBED, N_PIECE_PAD), jnp.float32),
            pltpu.VMEM((EMBED, N_POS), jnp.float32),
            pltpu.VMEM((8 * TROW,), jnp.float32),
            pltpu.VMEM((8, BCHUNK), jnp.int32),
            pltpu.VMEM((8, BCHUNK), jnp.int32),
            pltpu.VMEM((EMBED, BCHUNK), jnp.float32),
            pltpu.VMEM((EMBED, BCHUNK), jnp.float32),
            pltpu.SemaphoreType.DMA,
            pltpu.SemaphoreType.DMA,
            pltpu.SemaphoreType.DMA,
            pltpu.SemaphoreType.DMA,
        ],
    )
    def k(board_hbm, piece_hbm, pos_hbm, out_hbm,
          pbuf, qbuf, tbuf, bba, bbb, ob0, ob1, os0, os1, bs0, bs1):
        wid = lax.axis_index("s") * 2 + lax.axis_index("c")
        r = wid // 4
        bq = wid % 4
        bbase = bq * BQUARTER

        off = (r // 2) * LANES
        pltpu.sync_copy(piece_hbm, pbuf)
        pltpu.sync_copy(pos_hbm, qbuf)
        lane0 = r * 8 - off
        lanes = [
            jnp.full((LANES,), 0, jnp.int32) + (lane0 + c) for c in range(8)
        ]

        def build_row(d, carry):
            piece_vec = pbuf[d, :]
            pos_vec = qbuf[d, pl.ds(off, LANES)]
            for c in range(8):
                tbuf[pl.ds(c * TROW + d * N_PIECE_PAD, LANES)] = (
                    piece_vec + _dyn_gather(pos_vec, lanes[c])
                )
            return carry

        lax.fori_loop(0, EMBED, build_row, 0)

        def fill_chunk(bb, ob, c):
            tvecs = [
                tbuf[pl.ds((c * EMBED + d) * N_PIECE_PAD, LANES)]
                for d in range(EMBED)
            ]

            @plsc.parallel_loop(0, BCHUNK // LANES, unroll=1)
            def _fill(kk):
                bv = bb[c, pl.ds(kk * LANES, LANES)]
                for d in range(EMBED):
                    ob[d, pl.ds(kk * LANES, LANES)] = _dyn_gather(tvecs[d], bv)

        def out_slice(c, g):
            return out_hbm.at[r, c, :, pl.ds(bbase + g * BCHUNK, BCHUNK)]

        obs = (ob0, ob1)
        sems = (os0, os1)
        bbs = (bba, bbb)
        bsems = (bs0, bs1)

        def board_slab(g):
            return board_hbm.at[r, :, pl.ds(bbase + g * BCHUNK, BCHUNK)]

        pltpu.async_copy(board_slab(0), bbs[0], bsems[0])
        for g in range(NCH):
            if g + 1 < NCH:
                pltpu.async_copy(board_slab(g + 1), bbs[(g + 1) % 2], bsems[(g + 1) % 2])
            pltpu.make_async_copy(board_slab(g), bbs[g % 2], bsems[g % 2]).wait()
            for c in range(8):
                buf = c % 2
                if g > 0 or c >= 2:
                    pltpu.make_async_copy(obs[buf], out_slice(0, 0), sems[buf]).wait()
                fill_chunk(bbs[g % 2], obs[buf], c)
                pltpu.async_copy(obs[buf], out_slice(c, g), sems[buf])
        pltpu.make_async_copy(ob0, out_slice(0, 0), os0).wait()
        pltpu.make_async_copy(ob1, out_slice(0, 0), os1).wait()

    return k(board_t, piece_pad_t, pos_t)


def kernel(board, piece_table, position_table):
    board_t = jnp.transpose(board.astype(jnp.int32), (1, 2, 0))
    piece_pad_t = jnp.pad(piece_table, ((0, N_PIECE_PAD - N_PIECE), (0, 0))).T
    pos_t = position_table.T
    out_t = _sc_lookup(board_t, piece_pad_t, pos_t)
    return jnp.transpose(out_t, (3, 0, 1, 2))


# speedup vs baseline: 1.2092x; 1.0727x over previous
import functools

import jax
import jax.numpy as jnp
from jax import lax
from jax.experimental import pallas as pl
from jax.experimental.pallas import tpu as pltpu
from jax.experimental.pallas import tpu_sc as plsc

EMBED = 32
N_PIECE = 13
N_PIECE_PAD = 16
N_POS = 64
BATCH = 16384
BCHUNK = 1024
BQUARTER = BATCH // 4
NCH = BQUARTER // BCHUNK
LANES = 16
TROW = EMBED * N_PIECE_PAD

_DNUMS = lax.GatherDimensionNumbers(
    offset_dims=(), collapsed_slice_dims=(0,), start_index_map=(0,)
)


def _dyn_gather(vec, idx):
    return lax.gather(
        vec,
        idx.reshape(LANES, 1),
        dimension_numbers=_DNUMS,
        slice_sizes=(1,),
        mode=lax.GatherScatterMode.PROMISE_IN_BOUNDS,
    )


def _sc_lookup(board_t, piece_pad_t, pos_t):
    mesh = plsc.VectorSubcoreMesh(core_axis_name="c", subcore_axis_name="s")

    @functools.partial(
        pl.kernel,
        out_type=jax.ShapeDtypeStruct((8, 8, EMBED, BATCH), jnp.float32),
        mesh=mesh,
        compiler_params=pltpu.CompilerParams(use_tc_tiling_on_sc=True),
        scratch_types=[
            pltpu.VMEM((EMBED, N_PIECE_PAD), jnp.float32),
            pltpu.VMEM((EMBED, N_POS), jnp.float32),
            pltpu.VMEM((8 * TROW,), jnp.float32),
            pltpu.VMEM((8, BCHUNK), jnp.int32),
            pltpu.VMEM((EMBED, BCHUNK), jnp.float32),
            pltpu.VMEM((EMBED, BCHUNK), jnp.float32),
            pltpu.SemaphoreType.DMA,
            pltpu.SemaphoreType.DMA,
            pltpu.SemaphoreType.DMA,
        ],
    )
    def k(board_hbm, piece_hbm, pos_hbm, out_hbm,
          pbuf, qbuf, tbuf, bb, ob0, ob1, os0, os1, bsem):
        wid = lax.axis_index("s") * 2 + lax.axis_index("c")
        r = wid // 4
        bq = wid % 4
        bbase = bq * BQUARTER

        def board_slab(g):
            return board_hbm.at[r, :, pl.ds(bbase + g * BCHUNK, BCHUNK)]

        pltpu.async_copy(board_slab(0), bb, bsem)

        off = (r // 2) * LANES
        pltpu.sync_copy(piece_hbm, pbuf)
        pltpu.sync_copy(pos_hbm, qbuf)
        lane0 = r * 8 - off
        lanes = [
            jnp.full((LANES,), 0, jnp.int32) + (lane0 + c) for c in range(8)
        ]

        def build_row(d, carry):
            piece_vec = pbuf[d, :]
            pos_vec = qbuf[d, pl.ds(off, LANES)]
            for c in range(8):
                tbuf[pl.ds(c * TROW + d * N_PIECE_PAD, LANES)] = (
                    piece_vec + _dyn_gather(pos_vec, lanes[c])
                )
            return carry

        lax.fori_loop(0, EMBED, build_row, 0)

        def fill_chunk(ob, c):
            tvecs = [
                tbuf[pl.ds((c * EMBED + d) * N_PIECE_PAD, LANES)]
                for d in range(EMBED)
            ]

            @plsc.parallel_loop(0, BCHUNK // LANES, unroll=1)
            def _fill(kk):
                bv = bb[c, pl.ds(kk * LANES, LANES)]
                for d in range(EMBED):
                    ob[d, pl.ds(kk * LANES, LANES)] = _dyn_gather(tvecs[d], bv)

        def out_slice(c, g):
            return out_hbm.at[r, c, :, pl.ds(bbase + g * BCHUNK, BCHUNK)]

        obs = (ob0, ob1)
        sems = (os0, os1)

        def loop_body(g, carry):
            @pl.when(g == 0)
            def _():
                pltpu.make_async_copy(board_slab(0), bb, bsem).wait()

            @pl.when(g > 0)
            def _():
                pltpu.sync_copy(board_slab(g), bb)
            for c in range(8):
                buf = c % 2

                def _wait():
                    pltpu.make_async_copy(obs[buf], out_slice(0, 0), sems[buf]).wait()

                if c < 2:
                    pl.when(g > 0)(_wait)
                else:
                    _wait()
                fill_chunk(obs[buf], c)
                pltpu.async_copy(obs[buf], out_slice(c, g), sems[buf])
            return carry

        lax.fori_loop(0, NCH, loop_body, 0)
        pltpu.make_async_copy(ob0, out_slice(0, 0), os0).wait()
        pltpu.make_async_copy(ob1, out_slice(0, 0), os1).wait()

    return k(board_t, piece_pad_t, pos_t)


def kernel(board, piece_table, position_table):
    board_t = jnp.transpose(board.astype(jnp.int32), (1, 2, 0))
    piece_pad_t = jnp.pad(piece_table, ((0, N_PIECE_PAD - N_PIECE), (0, 0))).T
    pos_t = position_table.T
    out_t = _sc_lookup(board_t, piece_pad_t, pos_t)
    return jnp.transpose(out_t, (3, 0, 1, 2))
